# baseline (speedup 1.0000x reference)
import functools

import jax
import jax.numpy as jnp
from jax import lax
from jax.experimental import pallas as pl
from jax.experimental.pallas import tpu as pltpu
from jax.experimental.pallas import tpu_sc as plsc

_NW = 32
_CHUNK = 800


def kernel(x, weight):
    b, h = x.shape
    n = b * h
    d = weight.shape[1]
    b_per_w = n // _NW
    rows_per_w = b // _NW
    nchunk = b_per_w // _CHUNK
    brow_per_chunk = _CHUNK // h

    mesh = plsc.VectorSubcoreMesh(core_axis_name="c", subcore_axis_name="s")

    @functools.partial(
        pl.kernel,
        out_type=jax.ShapeDtypeStruct((b, h, d), weight.dtype),
        mesh=mesh,
        compiler_params=pltpu.CompilerParams(use_tc_tiling_on_sc=False),
        scratch_types=[
            pltpu.VMEM((b_per_w,), jnp.int32),
            pltpu.VMEM((2, _CHUNK, d), jnp.float32),
            pltpu.SemaphoreType.DMA,
            pltpu.SemaphoreType.DMA,
            pltpu.SemaphoreType.DMA,
            pltpu.SemaphoreType.DMA,
        ],
    )
    def k(w_hbm, i_hbm, o_hbm, idx_v, rows_v, g0, g1, o0, o1):
        gsem = (g0, g1)
        osem = (o0, o1)
        wid = lax.axis_index("s") * 2 + lax.axis_index("c")
        base = wid * b_per_w
        brow_base = wid * rows_per_w
        pltpu.sync_copy(i_hbm.at[pl.ds(base, b_per_w)], idx_v)

        gh = [None, None]
        oh = [[] for _ in range(2)]

        def fire_gather(c):
            buf = c % 2
            gh[buf] = pltpu.async_copy(
                w_hbm.at[idx_v.at[pl.ds(c * _CHUNK, _CHUNK)]],
                rows_v.at[buf],
                gsem[buf],
            )

        fire_gather(0)
        for c in range(nchunk):
            buf = c % 2
            if c + 1 < nchunk:
                nbuf = (c + 1) % 2
                for hdl in oh[nbuf]:
                    hdl.wait()
                oh[nbuf] = []
                fire_gather(c + 1)
            gh[buf].wait()
            for r in range(brow_per_chunk):
                oh[buf].append(
                    pltpu.async_copy(
                        rows_v.at[buf, pl.ds(r * h, h)],
                        o_hbm.at[brow_base + c * brow_per_chunk + r],
                        osem[buf],
                    )
                )
        for buf in range(2):
            for hdl in oh[buf]:
                hdl.wait()

    return k(weight, x.reshape(n).astype(jnp.int32))


# speedup vs baseline: 1.0484x; 1.0484x over previous
import functools

import jax
import jax.numpy as jnp
from jax import lax
from jax.experimental import pallas as pl
from jax.experimental.pallas import tpu as pltpu
from jax.experimental.pallas import tpu_sc as plsc

_NW = 32
_RING = 5


def kernel(x, weight):
    b, h = x.shape
    n = b * h
    d = weight.shape[1]
    b_per_w = n // _NW
    rows_per_w = b // _NW

    mesh = plsc.VectorSubcoreMesh(core_axis_name="c", subcore_axis_name="s")

    @functools.partial(
        pl.kernel,
        out_type=jax.ShapeDtypeStruct((h, b, d), weight.dtype),
        mesh=mesh,
        compiler_params=pltpu.CompilerParams(
            use_tc_tiling_on_sc=False, needs_layout_passes=False
        ),
        scratch_types=[
            pltpu.VMEM((b_per_w,), jnp.int32),
            pltpu.VMEM((h * rows_per_w,), jnp.int32),
            pltpu.VMEM((_RING, rows_per_w, d), jnp.float32),
            pltpu.SemaphoreType.DMA,
        ]
        + [pltpu.SemaphoreType.DMA for _ in range(2 * _RING)],
    )
    def k(w_hbm, i_hbm, o_hbm, idx_v, idxt_v, rows_v, isem, *sems):
        gsem = sems[:_RING]
        osem = sems[_RING:]
        wid = lax.axis_index("s") * 2 + lax.axis_index("c")
        base = wid * b_per_w
        brow0 = wid * rows_per_w
        pltpu.sync_copy(i_hbm.at[pl.ds(base, b_per_w)], idx_v)

        lanes = jax.lax.iota(jnp.int32, 16)

        @pl.loop(0, h)
        def _(t):
            for jj in range(rows_per_w // 16):
                vals = plsc.load_gather(idx_v, [(jj * 16 + lanes) * h + t])
                idxt_v[pl.ds(pl.multiple_of(t * rows_per_w, 128) + jj * 16, 16)] = vals

        gh = [None] * _RING
        oh = [None] * _RING

        def fire_gather(t):
            j = t % _RING
            gh[j] = pltpu.async_copy(
                w_hbm.at[idxt_v.at[pl.ds(t * rows_per_w, rows_per_w)]],
                rows_v.at[j],
                gsem[j],
            )

        for t in range(h):
            j = t % _RING
            if t < _RING:
                fire_gather(t)
            gh[j].wait()
            oh[j] = pltpu.async_copy(
                rows_v.at[j], o_hbm.at[t, pl.ds(brow0, rows_per_w)], osem[j]
            )
            if t + _RING < h:
                oh[j].wait()
                fire_gather(t + _RING)
        for t in range(max(0, h - _RING), h):
            oh[t % _RING].wait()

    out_t = k(weight, x.reshape(n).astype(jnp.int32))
    return jnp.transpose(out_t, (1, 0, 2))
